# Initial kernel scaffold; baseline (speedup 1.0000x reference)
import functools

import jax
import jax.numpy as jnp
from jax import lax
from jax.experimental import pallas as pl
from jax.experimental.pallas import tpu as pltpu
from jax.experimental.pallas import tpu_sc as plsc

_N = 10000
_E = 320000
_H = 128
_G = 128

_NC = 2
_NS = 16
_NW = _NC * _NS
_CH = 128
_STEPS = -(-_E // (_NW * _CH))
_EPW = _CH * _STEPS
_EPAD = _NW * _EPW
_NPAD = 10240
_RPT = _NPAD // _NS
_ZCH = _RPT // _CH
_CW = 16


def _fill_2d(ref, nrows, value):
    ncol = ref.shape[1] // 16
    vec = jnp.full((16,), value, jnp.float32)

    def row(i, _):
        def col(j, _):
            ref[i, pl.ds(j * 16, 16)] = vec
            return 0
        return lax.fori_loop(0, ncol, col, 0)

    lax.fori_loop(0, nrows, row, 0)


def _seg_body(with_counts, p_hbm, src_hbm, dst_hbm, *rest):
    if with_counts:
        (sout_hbm, cout_hbm, src_v, dst_v, rows_v, cnt_v, acc, ccnt,
         sem) = rest
    else:
        (sout_hbm, src_v, dst_v, rows_v, acc, sem) = rest
    c = lax.axis_index("c")
    s = lax.axis_index("s")
    wid = s * _NC + c
    row0 = s * _RPT

    _fill_2d(rows_v, _CH, 0.0)
    for k in range(_ZCH):
        pltpu.sync_copy(rows_v, acc.at[pl.ds(row0 + k * _CH, _CH)])
    if with_counts:
        _fill_2d(cnt_v, _CH, 0.0)
        for k in range(_ZCH):
            pltpu.sync_copy(cnt_v, ccnt.at[pl.ds(row0 + k * _CH, _CH)])
        _fill_2d(cnt_v, _CH, 1.0)
    plsc.subcore_barrier()

    ebase = wid * _EPW

    def step(i, _):
        off = ebase + i * _CH
        pltpu.sync_copy(src_hbm.at[pl.ds(off, _CH)], src_v)
        pltpu.sync_copy(dst_hbm.at[pl.ds(off, _CH)], dst_v)
        pltpu.async_copy(p_hbm.at[src_v], rows_v, sem).wait()
        pltpu.sync_copy(rows_v, acc.at[dst_v], add=True)
        if with_counts:
            pltpu.sync_copy(cnt_v, ccnt.at[dst_v], add=True)
        return 0

    lax.fori_loop(0, _STEPS, step, 0)
    plsc.subcore_barrier()

    obase = c * _NPAD + row0
    pltpu.sync_copy(acc.at[pl.ds(row0, _RPT)], sout_hbm.at[pl.ds(obase, _RPT)])
    if with_counts:
        pltpu.sync_copy(ccnt.at[pl.ds(row0, _RPT)],
                        cout_hbm.at[pl.ds(obase, _RPT)])


def _make_seg(with_counts):
    out_type = [jax.ShapeDtypeStruct((_NC * _NPAD, _H), jnp.float32)]
    scratch = [
        pltpu.VMEM((_CH,), jnp.int32),
        pltpu.VMEM((_CH,), jnp.int32),
        pltpu.VMEM((_CH, _H), jnp.float32),
    ]
    if with_counts:
        out_type.append(jax.ShapeDtypeStruct((_NC * _NPAD, _CW), jnp.float32))
        scratch.append(pltpu.VMEM((_CH, _CW), jnp.float32))
    scratch.append(pltpu.VMEM_SHARED((_NPAD, _H), jnp.float32))
    if with_counts:
        scratch.append(pltpu.VMEM_SHARED((_NPAD, _CW), jnp.float32))
    scratch.append(pltpu.SemaphoreType.DMA)
    mesh = plsc.VectorSubcoreMesh(core_axis_name="c", subcore_axis_name="s")
    return pl.kernel(
        functools.partial(_seg_body, with_counts),
        out_type=tuple(out_type) if with_counts else out_type[0],
        mesh=mesh,
        scratch_types=scratch,
        name="seg_sum_counts" if with_counts else "seg_sum",
    )


_seg_with_counts = _make_seg(True)
_seg_no_counts = _make_seg(False)


def _pre_body(x_ref, wl_ref, wr_ref, bl_ref, p_ref, w_ref):
    x = x_ref[...]
    p_ref[...] = jnp.dot(x, wl_ref[...], preferred_element_type=jnp.float32)
    w_ref[...] = (jnp.dot(x, wr_ref[...], preferred_element_type=jnp.float32)
                  + bl_ref[...])


_tc_pre = pl.pallas_call(
    _pre_body,
    out_shape=(jax.ShapeDtypeStruct((_N, _H), jnp.float32),
               jax.ShapeDtypeStruct((_N, _H), jnp.float32)),
)


def _bn_relu(s_ref, c_ref, w_ref, g_ref, be_ref):
    s = s_ref[0:_N, :] + s_ref[_NPAD:_NPAD + _N, :]
    cnt = c_ref[0:_N, 0:1] + c_ref[_NPAD:_NPAD + _N, 0:1]
    agg = s * (1.0 / jnp.maximum(cnt, 1.0))
    z = agg + w_ref[...]
    m = jnp.mean(z, axis=0, keepdims=True)
    zc = z - m
    v = jnp.mean(zc * zc, axis=0, keepdims=True)
    return jnp.maximum(zc * lax.rsqrt(v + 1e-5) * g_ref[...] + be_ref[...],
                       0.0)


def _mid_body(s_ref, c_ref, w_ref, g_ref, be_ref, wl_ref, wr_ref, bl_ref,
              p_ref, wn_ref):
    h = _bn_relu(s_ref, c_ref, w_ref, g_ref, be_ref)
    p_ref[...] = jnp.dot(h, wl_ref[...], preferred_element_type=jnp.float32)
    wn_ref[...] = (jnp.dot(h, wr_ref[...], preferred_element_type=jnp.float32)
                   + bl_ref[...])


_tc_mid = pl.pallas_call(
    _mid_body,
    out_shape=(jax.ShapeDtypeStruct((_N, _H), jnp.float32),
               jax.ShapeDtypeStruct((_N, _H), jnp.float32)),
)


def _fin_body(s_ref, c_ref, w_ref, g_ref, be_ref, b_ref, wh1_ref, bh1_ref,
              wh2_ref, bh2_ref, o_ref):
    h = _bn_relu(s_ref, c_ref, w_ref, g_ref, be_ref)
    onehot = (b_ref[...] == lax.broadcasted_iota(jnp.int32, (1, _G), 1)
              ).astype(jnp.float32)
    hg_sum = lax.dot_general(onehot, h, (((0,), (0,)), ((), ())),
                             preferred_element_type=jnp.float32)
    cg = jnp.sum(onehot, axis=0)[:, None]
    hg = hg_sum * (1.0 / jnp.maximum(cg, 1.0))
    t = jnp.maximum(
        jnp.dot(hg, wh1_ref[...], preferred_element_type=jnp.float32)
        + bh1_ref[...], 0.0)
    o_ref[...] = (jnp.dot(t, wh2_ref[...], preferred_element_type=jnp.float32)
                  + bh2_ref[...])


_tc_fin = pl.pallas_call(
    _fin_body,
    out_shape=jax.ShapeDtypeStruct((_G, 1), jnp.float32),
)


def kernel(x, edge_index, batch, Wl0, bl0, Wr0, g0, be0, Wl1, bl1, Wr1, g1,
           be1, Wl2, bl2, Wr2, g2, be2, Wh1, bh1, Wh2, bh2):
    pad = _EPAD - _E
    src = jnp.concatenate([edge_index[0], jnp.zeros((pad,), jnp.int32)])
    dst = jnp.concatenate([edge_index[1], jnp.full((pad,), _N, jnp.int32)])
    b2 = batch.reshape(_N, 1)
    bl0r = bl0.reshape(1, _H)
    bl1r = bl1.reshape(1, _H)
    bl2r = bl2.reshape(1, _H)
    g0r, be0r = g0.reshape(1, _H), be0.reshape(1, _H)
    g1r, be1r = g1.reshape(1, _H), be1.reshape(1, _H)
    g2r, be2r = g2.reshape(1, _H), be2.reshape(1, _H)
    bh1r = bh1.reshape(1, _H // 2)
    bh2r = bh2.reshape(1, 1)

    p0, w0 = _tc_pre(x, Wl0, Wr0, bl0r)
    s0, cp = _seg_with_counts(p0, src, dst)
    p1, w1 = _tc_mid(s0, cp, w0, g0r, be0r, Wl1, Wr1, bl1r)
    s1 = _seg_no_counts(p1, src, dst)
    p2, w2 = _tc_mid(s1, cp, w1, g1r, be1r, Wl2, Wr2, bl2r)
    s2 = _seg_no_counts(p2, src, dst)
    out = _tc_fin(s2, cp, w2, g2r, be2r, b2, Wh1, bh1r, Wh2, bh2r)
    return out


# speedup vs baseline: 3.8562x; 3.8562x over previous
import jax
import jax.numpy as jnp
from jax import lax
from jax.experimental import pallas as pl
from jax.experimental.pallas import tpu as pltpu
from jax.experimental.pallas import tpu_sc as plsc

_N = 10000
_E = 320000
_H = 128
_G = 128

_NC = 2
_NS = 16
_NW = _NC * _NS
_CH = 128
_STEPS = -(-_E // (_NW * _CH))
_EPW = _CH * _STEPS
_EPAD = _NW * _EPW
_NPAD = 10240
_RPT = _NPAD // _NS
_ZCH = _RPT // _CH


def _fill_2d(ref, nrows, value):
    ncol = ref.shape[1] // 16
    vec = jnp.full((16,), value, jnp.float32)

    def row(i, _):
        def col(j, _):
            ref[i, pl.ds(j * 16, 16)] = vec
            return 0
        return lax.fori_loop(0, ncol, col, 0)

    lax.fori_loop(0, nrows, row, 0)


def _seg_body(p_hbm, src_hbm, dst_hbm, sout_hbm, src_v, dst_v, rows_v, acc,
              sem):
    c = lax.axis_index("c")
    s = lax.axis_index("s")
    wid = s * _NC + c
    row0 = s * _RPT

    _fill_2d(rows_v, _CH, 0.0)
    for k in range(_ZCH):
        pltpu.sync_copy(rows_v, acc.at[pl.ds(row0 + k * _CH, _CH)])
    plsc.subcore_barrier()

    ebase = wid * _EPW

    def step(i, _):
        off = ebase + i * _CH
        pltpu.sync_copy(src_hbm.at[pl.ds(off, _CH)], src_v)
        pltpu.sync_copy(dst_hbm.at[pl.ds(off, _CH)], dst_v)
        pltpu.async_copy(p_hbm.at[src_v], rows_v, sem).wait()
        pltpu.sync_copy(rows_v, acc.at[dst_v], add=True)
        return 0

    lax.fori_loop(0, _STEPS, step, 0)
    plsc.subcore_barrier()

    obase = c * _NPAD + row0
    pltpu.sync_copy(acc.at[pl.ds(row0, _RPT)], sout_hbm.at[pl.ds(obase, _RPT)])


_seg_sum = pl.kernel(
    _seg_body,
    out_type=jax.ShapeDtypeStruct((_NC * _NPAD, _H), jnp.float32),
    mesh=plsc.VectorSubcoreMesh(core_axis_name="c", subcore_axis_name="s"),
    scratch_types=[
        pltpu.VMEM((_CH,), jnp.int32),
        pltpu.VMEM((_CH,), jnp.int32),
        pltpu.VMEM((_CH, _H), jnp.float32),
        pltpu.VMEM_SHARED((_NPAD, _H), jnp.float32),
        pltpu.SemaphoreType.DMA,
    ],
    name="seg_sum",
)


def _cnt_body(dst_hbm, cout_hbm, dst_v, ones_v, acc, sem):
    c = lax.axis_index("c")
    s = lax.axis_index("s")
    wid = s * _NC + c
    row0 = s * _RPT

    _fill_2d(ones_v, _CH, 0.0)
    for k in range(_ZCH):
        pltpu.sync_copy(ones_v, acc.at[pl.ds(row0 + k * _CH, _CH)])
    _fill_2d(ones_v, _CH, 1.0)
    plsc.subcore_barrier()

    ebase = wid * _EPW

    def step(i, _):
        off = ebase + i * _CH
        pltpu.sync_copy(dst_hbm.at[pl.ds(off, _CH)], dst_v)
        pltpu.sync_copy(ones_v, acc.at[dst_v], add=True)
        return 0

    lax.fori_loop(0, _STEPS, step, 0)
    plsc.subcore_barrier()

    obase = c * _NPAD + row0
    pltpu.sync_copy(acc.at[pl.ds(row0, _RPT)], cout_hbm.at[pl.ds(obase, _RPT)])


_cnt_sum = pl.kernel(
    _cnt_body,
    out_type=jax.ShapeDtypeStruct((_NC * _NPAD, _H), jnp.float32),
    mesh=plsc.VectorSubcoreMesh(core_axis_name="c", subcore_axis_name="s"),
    scratch_types=[
        pltpu.VMEM((_CH,), jnp.int32),
        pltpu.VMEM((_CH, _H), jnp.float32),
        pltpu.VMEM_SHARED((_NPAD, _H), jnp.float32),
        pltpu.SemaphoreType.DMA,
    ],
    name="cnt_sum",
)


def _cnt_prep_body(c_ref, inv_ref):
    cnt = c_ref[0:_N, 0:1] + c_ref[_NPAD:_NPAD + _N, 0:1]
    inv_ref[...] = 1.0 / jnp.maximum(cnt, 1.0)


_tc_cnt_prep = pl.pallas_call(
    _cnt_prep_body,
    out_shape=jax.ShapeDtypeStruct((_N, 1), jnp.float32),
)


def _pre_body(x_ref, wl_ref, wr_ref, bl_ref, p_ref, w_ref):
    x = x_ref[...]
    p_ref[...] = jnp.dot(x, wl_ref[...], preferred_element_type=jnp.float32)
    w_ref[...] = (jnp.dot(x, wr_ref[...], preferred_element_type=jnp.float32)
                  + bl_ref[...])


_tc_pre = pl.pallas_call(
    _pre_body,
    out_shape=(jax.ShapeDtypeStruct((_N, _H), jnp.float32),
               jax.ShapeDtypeStruct((_N, _H), jnp.float32)),
)


def _bn_relu(s_ref, inv_ref, w_ref, g_ref, be_ref):
    s = s_ref[0:_N, :] + s_ref[_NPAD:_NPAD + _N, :]
    agg = s * inv_ref[...]
    z = agg + w_ref[...]
    m = jnp.mean(z, axis=0, keepdims=True)
    zc = z - m
    v = jnp.mean(zc * zc, axis=0, keepdims=True)
    return jnp.maximum(zc * lax.rsqrt(v + 1e-5) * g_ref[...] + be_ref[...],
                       0.0)


def _mid_body(s_ref, inv_ref, w_ref, g_ref, be_ref, wl_ref, wr_ref, bl_ref,
              p_ref, wn_ref):
    h = _bn_relu(s_ref, inv_ref, w_ref, g_ref, be_ref)
    p_ref[...] = jnp.dot(h, wl_ref[...], preferred_element_type=jnp.float32)
    wn_ref[...] = (jnp.dot(h, wr_ref[...], preferred_element_type=jnp.float32)
                   + bl_ref[...])


_tc_mid = pl.pallas_call(
    _mid_body,
    out_shape=(jax.ShapeDtypeStruct((_N, _H), jnp.float32),
               jax.ShapeDtypeStruct((_N, _H), jnp.float32)),
)


def _fin_body(s_ref, inv_ref, w_ref, g_ref, be_ref, b_ref, wh1_ref, bh1_ref,
              wh2_ref, bh2_ref, o_ref):
    h = _bn_relu(s_ref, inv_ref, w_ref, g_ref, be_ref)
    onehot = (b_ref[...] == lax.broadcasted_iota(jnp.int32, (1, _G), 1)
              ).astype(jnp.float32)
    hg_sum = lax.dot_general(onehot, h, (((0,), (0,)), ((), ())),
                             preferred_element_type=jnp.float32)
    cg = jnp.sum(onehot, axis=0)[:, None]
    hg = hg_sum * (1.0 / jnp.maximum(cg, 1.0))
    t = jnp.maximum(
        jnp.dot(hg, wh1_ref[...], preferred_element_type=jnp.float32)
        + bh1_ref[...], 0.0)
    o_ref[...] = (jnp.dot(t, wh2_ref[...], preferred_element_type=jnp.float32)
                  + bh2_ref[...])


_tc_fin = pl.pallas_call(
    _fin_body,
    out_shape=jax.ShapeDtypeStruct((_G, 1), jnp.float32),
)


def kernel(x, edge_index, batch, Wl0, bl0, Wr0, g0, be0, Wl1, bl1, Wr1, g1,
           be1, Wl2, bl2, Wr2, g2, be2, Wh1, bh1, Wh2, bh2):
    pad = _EPAD - _E
    src = jnp.concatenate([edge_index[0], jnp.zeros((pad,), jnp.int32)])
    dst = jnp.concatenate([edge_index[1], jnp.full((pad,), _N, jnp.int32)])
    b2 = batch.reshape(_N, 1)
    bl0r = bl0.reshape(1, _H)
    bl1r = bl1.reshape(1, _H)
    bl2r = bl2.reshape(1, _H)
    g0r, be0r = g0.reshape(1, _H), be0.reshape(1, _H)
    g1r, be1r = g1.reshape(1, _H), be1.reshape(1, _H)
    g2r, be2r = g2.reshape(1, _H), be2.reshape(1, _H)
    bh1r = bh1.reshape(1, _H // 2)
    bh2r = bh2.reshape(1, 1)

    craw = _cnt_sum(dst)
    invc = _tc_cnt_prep(craw)
    p0, w0 = _tc_pre(x, Wl0, Wr0, bl0r)
    s0 = _seg_sum(p0, src, dst)
    p1, w1 = _tc_mid(s0, invc, w0, g0r, be0r, Wl1, Wr1, bl1r)
    s1 = _seg_sum(p1, src, dst)
    p2, w2 = _tc_mid(s1, invc, w1, g1r, be1r, Wl2, Wr2, bl2r)
    s2 = _seg_sum(p2, src, dst)
    out = _tc_fin(s2, invc, w2, g2r, be2r, b2, Wh1, bh1r, Wh2, bh2r)
    return out
